# baseline (speedup 1.0000x reference)
import jax
import jax.numpy as jnp
from jax import lax
from jax.experimental import pallas as pl
from jax.experimental.pallas import tpu as pltpu
from jax.experimental.pallas import tpu_sc as plsc

N = 100000
D = 128
P = 50000
L = 16
NC, NS = 2, 16
NW = NC * NS
CHUNK = 56
NCHUNK = 28
BW = CHUNK * NCHUNK
BLK = NC * BW
OVERLAP = NS * BLK - P


def _body(ia0, ib0, ia1, ib1, ia2, ib2, t_aig, t_mig, t_xmg, t_xag, out,
          ia0_v, ib0_v, ia1_v, ib1_v, ia2_v, ib2_v,
          rA1, rA2, rB1, rB2, acc_v, semA, semB):
    c = lax.axis_index("c")
    s = lax.axis_index("s")
    wid = s * NC + c
    is_tail = jnp.logical_and(s == NS - 1, c == NC - 1)
    base = jnp.where(is_tail, P - BW, s * BLK + c * BW)

    idx_v = ((ia0_v, ib0_v), (ia1_v, ib1_v), (ia2_v, ib2_v))

    def stage_idx(p, ia, ib, sem):
        cp1 = pltpu.async_copy(ia.at[pl.ds(base, BW)], idx_v[p][0], sem)
        cp2 = pltpu.async_copy(ib.at[pl.ds(base, BW)], idx_v[p][1], sem)
        cp1.wait()
        cp2.wait()

        @pl.when(is_tail)
        def _():
            z = jnp.zeros((L,), jnp.int32)
            for k in range(OVERLAP // L):
                idx_v[p][0][pl.ds(k * L, L)] = z
                idx_v[p][1][pl.ds(k * L, L)] = z

    def start(p, cc, r1, r2, sem, tab2):
        pltpu.async_copy(t_aig.at[idx_v[p][0].at[pl.ds(cc * CHUNK, CHUNK)]],
                         r1, sem)
        pltpu.async_copy(tab2.at[idx_v[p][1].at[pl.ds(cc * CHUNK, CHUNK)]],
                         r2, sem)

    def drain(r1, r2, sem):
        pltpu.make_async_copy(t_aig.at[ia0_v.at[pl.ds(0, CHUNK)]],
                              r1, sem).wait()
        pltpu.make_async_copy(t_aig.at[ib0_v.at[pl.ds(0, CHUNK)]],
                              r2, sem).wait()

    def compute(r1, r2, acc):
        @pl.loop(0, CHUNK, init_carry=acc, unroll=4)
        def _rows(r, acc):
            a0, a1 = acc
            t = [jnp.abs(r1[r, pl.ds(k * L, L)] - r2[r, pl.ds(k * L, L)])
                 for k in range(D // L)]
            cs = ((t[0] + t[1]) + (t[2] + t[3])) + ((t[4] + t[5]) + (t[6] + t[7]))
            return (a1, a0 + cs)
        return _rows

    acc = (jnp.zeros((L,), jnp.float32), jnp.zeros((L,), jnp.float32))
    f_pad = jnp.where(is_tail, jnp.float32(OVERLAP), jnp.float32(0.0))
    tabs2 = (t_mig, t_xmg, t_xag)

    stage_idx(0, ia0, ib0, semA)
    for p, tab2 in enumerate(tabs2):
        if p == 0:
            start(p, 0, rA1, rA2, semA, tab2)
            stage_idx(1, ia1, ib1, semB)
            stage_idx(2, ia2, ib2, semB)
        drain(rA1, rA2, semA)
        t = [jnp.abs(rA1[0, pl.ds(k * L, L)] - rA2[0, pl.ds(k * L, L)])
             for k in range(D // L)]
        v0 = ((t[0] + t[1]) + (t[2] + t[3])) + ((t[4] + t[5]) + (t[6] + t[7]))
        start(p, 1, rB1, rB2, semB, tab2)

        @pl.loop(0, (NCHUNK - 2) // 2, init_carry=acc)
        def _chunks(t, acc, p=p, tab2=tab2):
            cc = 2 * t
            acc = compute(rA1, rA2, acc)
            start(p, cc + 2, rA1, rA2, semA, tab2)
            drain(rB1, rB2, semB)
            acc = compute(rB1, rB2, acc)
            start(p, cc + 3, rB1, rB2, semB, tab2)
            drain(rA1, rA2, semA)
            return acc

        acc = _chunks
        acc = compute(rA1, rA2, acc)
        if p < 2:
            start(p + 1, 0, rA1, rA2, semA, tabs2[p + 1])
        drain(rB1, rB2, semB)
        acc = compute(rB1, rB2, acc)
        acc = (acc[0] - f_pad * v0, acc[1])

    acc_v[...] = acc[0] + acc[1]
    pltpu.sync_copy(acc_v, out.at[wid])


@jax.jit
def _run(ia0, ib0, ia1, ib1, ia2, ib2, t_aig, t_mig, t_xmg, t_xag):
    mesh = plsc.VectorSubcoreMesh(core_axis_name="c", subcore_axis_name="s")
    f = pl.kernel(
        _body,
        out_type=jax.ShapeDtypeStruct((NW, L), jnp.float32),
        mesh=mesh,
        scratch_types=[
            pltpu.VMEM((BW,), jnp.int32),
            pltpu.VMEM((BW,), jnp.int32),
            pltpu.VMEM((BW,), jnp.int32),
            pltpu.VMEM((BW,), jnp.int32),
            pltpu.VMEM((BW,), jnp.int32),
            pltpu.VMEM((BW,), jnp.int32),
            pltpu.VMEM((CHUNK, D), jnp.float32),
            pltpu.VMEM((CHUNK, D), jnp.float32),
            pltpu.VMEM((CHUNK, D), jnp.float32),
            pltpu.VMEM((CHUNK, D), jnp.float32),
            pltpu.VMEM((L,), jnp.float32),
            pltpu.SemaphoreType.DMA,
            pltpu.SemaphoreType.DMA,
        ],
    )
    partials = f(ia0, ib0, ia1, ib1, ia2, ib2, t_aig, t_mig, t_xmg, t_xag)
    return jnp.sum(partials) / jnp.float32(P * D)


def kernel(aig_hf, mig_hf, xmg_hf, xag_hf,
           aig_mig_equ, mig_aig_equ,
           aig_xmg_equ, xmg_aig_equ,
           aig_xag_equ, xag_aig_equ):
    return _run(aig_mig_equ.astype(jnp.int32), mig_aig_equ.astype(jnp.int32),
                aig_xmg_equ.astype(jnp.int32), xmg_aig_equ.astype(jnp.int32),
                aig_xag_equ.astype(jnp.int32), xag_aig_equ.astype(jnp.int32),
                aig_hf, mig_hf, xmg_hf, xag_hf)


# speedup vs baseline: 1.0451x; 1.0451x over previous
import jax
import jax.numpy as jnp
from jax import lax
from jax.experimental import pallas as pl
from jax.experimental.pallas import tpu as pltpu
from jax.experimental.pallas import tpu_sc as plsc

N = 100000
D = 128
P = 50000
L = 16
NC, NS = 2, 16
NW = NC * NS
CHUNK = 56
NCHUNK = 28
BW = CHUNK * NCHUNK
BLK = NC * BW
OVERLAP = NS * BLK - P


def _body(ia0, ib0, ia1, ib1, ia2, ib2, t_aig, t_mig, t_xmg, t_xag, out,
          ia0_v, ib0_v, ia1_v, ib1_v, ia2_v, ib2_v,
          rA1, rA2, rB1, rB2, acc_v, semA, semB):
    c = lax.axis_index("c")
    s = lax.axis_index("s")
    wid = s * NC + c
    is_tail = jnp.logical_and(s == NS - 1, c == NC - 1)
    base = jnp.where(is_tail, P - BW, s * BLK + c * BW)

    idx_v = ((ia0_v, ib0_v), (ia1_v, ib1_v), (ia2_v, ib2_v))

    cps = []
    for p, (ia, ib) in enumerate(((ia0, ib0), (ia1, ib1), (ia2, ib2))):
        cps.append(pltpu.async_copy(ia.at[pl.ds(base, BW)], idx_v[p][0], semA))
        cps.append(pltpu.async_copy(ib.at[pl.ds(base, BW)], idx_v[p][1], semA))
    for cp in cps:
        cp.wait()

    @pl.when(is_tail)
    def _():
        z = jnp.zeros((L,), jnp.int32)
        for p in range(3):
            for k in range(OVERLAP // L):
                idx_v[p][0][pl.ds(k * L, L)] = z
                idx_v[p][1][pl.ds(k * L, L)] = z

    def start(p, cc, r1, r2, sem, tab2):
        pltpu.async_copy(t_aig.at[idx_v[p][0].at[pl.ds(cc * CHUNK, CHUNK)]],
                         r1, sem)
        pltpu.async_copy(tab2.at[idx_v[p][1].at[pl.ds(cc * CHUNK, CHUNK)]],
                         r2, sem)

    def drain(r1, r2, sem):
        pltpu.make_async_copy(t_aig.at[ia0_v.at[pl.ds(0, CHUNK)]],
                              r1, sem).wait()
        pltpu.make_async_copy(t_aig.at[ib0_v.at[pl.ds(0, CHUNK)]],
                              r2, sem).wait()

    def compute(r1, r2, acc):
        @pl.loop(0, CHUNK, init_carry=acc, unroll=4)
        def _rows(r, acc):
            a0, a1 = acc
            t = [jnp.abs(r1[r, pl.ds(k * L, L)] - r2[r, pl.ds(k * L, L)])
                 for k in range(D // L)]
            cs = ((t[0] + t[1]) + (t[2] + t[3])) + ((t[4] + t[5]) + (t[6] + t[7]))
            return (a1, a0 + cs)
        return _rows

    acc = (jnp.zeros((L,), jnp.float32), jnp.zeros((L,), jnp.float32))
    f_pad = jnp.where(is_tail, jnp.float32(OVERLAP), jnp.float32(0.0))
    tabs2 = (t_mig, t_xmg, t_xag)

    for p, tab2 in enumerate(tabs2):
        if p == 0:
            start(p, 0, rA1, rA2, semA, tab2)
        drain(rA1, rA2, semA)
        t = [jnp.abs(rA1[0, pl.ds(k * L, L)] - rA2[0, pl.ds(k * L, L)])
             for k in range(D // L)]
        v0 = ((t[0] + t[1]) + (t[2] + t[3])) + ((t[4] + t[5]) + (t[6] + t[7]))
        start(p, 1, rB1, rB2, semB, tab2)

        @pl.loop(0, (NCHUNK - 2) // 2, init_carry=acc)
        def _chunks(t, acc, p=p, tab2=tab2):
            cc = 2 * t
            acc = compute(rA1, rA2, acc)
            start(p, cc + 2, rA1, rA2, semA, tab2)
            drain(rB1, rB2, semB)
            acc = compute(rB1, rB2, acc)
            start(p, cc + 3, rB1, rB2, semB, tab2)
            drain(rA1, rA2, semA)
            return acc

        acc = _chunks
        acc = compute(rA1, rA2, acc)
        if p < 2:
            start(p + 1, 0, rA1, rA2, semA, tabs2[p + 1])
        drain(rB1, rB2, semB)
        acc = compute(rB1, rB2, acc)
        acc = (acc[0] - f_pad * v0, acc[1])

    acc_v[...] = acc[0] + acc[1]
    pltpu.sync_copy(acc_v, out.at[wid])


@jax.jit
def _run(ia0, ib0, ia1, ib1, ia2, ib2, t_aig, t_mig, t_xmg, t_xag):
    mesh = plsc.VectorSubcoreMesh(core_axis_name="c", subcore_axis_name="s")
    f = pl.kernel(
        _body,
        out_type=jax.ShapeDtypeStruct((NW, L), jnp.float32),
        mesh=mesh,
        scratch_types=[
            pltpu.VMEM((BW,), jnp.int32),
            pltpu.VMEM((BW,), jnp.int32),
            pltpu.VMEM((BW,), jnp.int32),
            pltpu.VMEM((BW,), jnp.int32),
            pltpu.VMEM((BW,), jnp.int32),
            pltpu.VMEM((BW,), jnp.int32),
            pltpu.VMEM((CHUNK, D), jnp.float32),
            pltpu.VMEM((CHUNK, D), jnp.float32),
            pltpu.VMEM((CHUNK, D), jnp.float32),
            pltpu.VMEM((CHUNK, D), jnp.float32),
            pltpu.VMEM((L,), jnp.float32),
            pltpu.SemaphoreType.DMA,
            pltpu.SemaphoreType.DMA,
        ],
    )
    partials = f(ia0, ib0, ia1, ib1, ia2, ib2, t_aig, t_mig, t_xmg, t_xag)
    return jnp.sum(partials) / jnp.float32(P * D)


def kernel(aig_hf, mig_hf, xmg_hf, xag_hf,
           aig_mig_equ, mig_aig_equ,
           aig_xmg_equ, xmg_aig_equ,
           aig_xag_equ, xag_aig_equ):
    return _run(aig_mig_equ.astype(jnp.int32), mig_aig_equ.astype(jnp.int32),
                aig_xmg_equ.astype(jnp.int32), xmg_aig_equ.astype(jnp.int32),
                aig_xag_equ.astype(jnp.int32), xag_aig_equ.astype(jnp.int32),
                aig_hf, mig_hf, xmg_hf, xag_hf)
